# Initial kernel scaffold; baseline (speedup 1.0000x reference)
import jax
import jax.numpy as jnp
from jax.experimental import pallas as pl


def kernel(x, edge_index, edge_attr, batch, W_in, b_in, ln1_g, ln1_b, basis_kernels, roi_community, W_e, b_e, conv_bias, ln2_g, ln2_b, Wa1, ba1, Wa2, ba2, Wg1, bg1, Wg2, bg2, Wh1, bh1, lnh_g, lnh_b, Wh2, bh2, Wh3, bh3):
    raise NotImplementedError("write your pallas kernel here")


# speedup vs baseline: 4.0092x; 4.0092x over previous
import functools

import jax
import jax.numpy as jnp
from jax import lax
from jax.experimental import pallas as pl
from jax.experimental.pallas import tpu as pltpu
from jax.experimental.pallas import tpu_sc as plsc

N = 9916
NG = 37
PER = 268
E = 317312
IN = 268
H = 128
C = 7
KP = 214

RP = 272
INP = 272
NP = NG * RP
ACCR = 10240
DUMMY = 10080

NW = 32
EB = 128
NB = 78
EP = NW * NB * EB

_NEG = -3.0e38


def _ln(x, g, b, eps=1e-5):
    m = x.mean(-1, keepdims=True)
    v = ((x - m) ** 2).mean(-1, keepdims=True)
    return (x - m) / jnp.sqrt(v + eps) * g + b


def _elu(x):
    return jnp.where(x > 0, x, jnp.exp(x) - 1.0)


def _sigmoid(x):
    return 1.0 / (1.0 + jnp.exp(-x))


def _pre_body(x_ref, win_ref, bin_ref, g1_ref, b1_ref, roi_ref, basis_ref,
              out_ref):
    xb = jnp.dot(x_ref[...], win_ref[...],
                 preferred_element_type=jnp.float32) + bin_ref[...]
    hb = _elu(_ln(xb, g1_ref[...], b1_ref[...]))
    roi = roi_ref[...]
    roi = roi - jnp.max(roi, -1, keepdims=True)
    er = jnp.exp(roi)
    cw = er / jnp.sum(er, -1, keepdims=True)
    acc = jnp.zeros((RP, H), jnp.float32)
    for c in range(C):
        acc = acc + jnp.dot(hb * cw[:, c:c + 1], basis_ref[c],
                            preferred_element_type=jnp.float32)
    out_ref[...] = acc


def _pre(x_p, win_p, b_in, ln1_g, ln1_b, roi_p, basis):
    return pl.pallas_call(
        _pre_body,
        grid=(NG,),
        in_specs=[
            pl.BlockSpec((RP, INP), lambda g: (g, 0)),
            pl.BlockSpec((INP, H), lambda g: (0, 0)),
            pl.BlockSpec((1, H), lambda g: (0, 0)),
            pl.BlockSpec((1, H), lambda g: (0, 0)),
            pl.BlockSpec((1, H), lambda g: (0, 0)),
            pl.BlockSpec((RP, C), lambda g: (0, 0)),
            pl.BlockSpec((C, H, H), lambda g: (0, 0, 0)),
        ],
        out_specs=pl.BlockSpec((RP, H), lambda g: (g, 0)),
        out_shape=jax.ShapeDtypeStruct((NP, H), jnp.float32),
    )(x_p, win_p, b_in, ln1_g, ln1_b, roi_p, basis)


def _sc_body(xt_hbm, pk_hbm, we_hbm, be_hbm,
             out_hbm,
             pk0, pk1, we_v, be_v, buf0, buf1, acc_sh,
             sem0, sem1, semp0, semp1):
    cid = lax.axis_index("c")
    sid = lax.axis_index("s")
    wid = cid * 16 + sid

    pltpu.sync_copy(we_hbm, we_v)
    pltpu.sync_copy(be_hbm, be_v)

    nw = [-(we_v[pl.ds(f * 16, 16)]) for f in range(H // 16)]
    nb = [-(be_v[pl.ds(f * 16, 16)]) for f in range(H // 16)]

    zero16 = jnp.zeros((16,), jnp.float32)

    def _zrow(i, _):
        for f in range(H // 16):
            buf0[i, pl.ds(f * 16, 16)] = zero16
        return 0

    lax.fori_loop(0, EB, _zrow, 0)
    for k in range(640 // EB):
        pltpu.sync_copy(buf0, acc_sh.at[pl.ds(sid * 640 + k * EB, EB)])
    plsc.subcore_barrier()

    def _apply_ew(buf, pk):
        def _edge(e, _):
            e16 = jnp.full((16,), e, jnp.int32)
            two16 = jnp.full((16,), 2, jnp.int32)
            eab = plsc.bitcast(plsc.load_gather(pk, [two16, e16]),
                               jnp.float32)
            for f in range(H // 16):
                s = 1.0 / (1.0 + jnp.exp(eab * nw[f] + nb[f]))
                buf[e, pl.ds(f * 16, 16)] = buf[e, pl.ds(f * 16, 16)] * s
            return 0

        lax.fori_loop(0, EB, _edge, 0)

    pltpu.sync_copy(pk_hbm.at[wid, 0], pk0)
    pltpu.sync_copy(pk_hbm.at[wid, 1], pk1)
    pltpu.async_copy(xt_hbm.at[pk0.at[0]], buf0, sem0)
    pltpu.async_copy(xt_hbm.at[pk1.at[0]], buf1, sem1)

    def _outer(k, _):
        j0 = 2 * k
        pltpu.make_async_copy(xt_hbm.at[pk0.at[0]], buf0, sem0).wait()
        _apply_ew(buf0, pk0)
        pltpu.sync_copy(buf0, acc_sh.at[pk0.at[1]], add=True)

        @pl.when(k < NB // 2 - 1)
        def _():
            pltpu.async_copy(pk_hbm.at[wid, j0 + 2], pk0, semp0)

        j1 = 2 * k + 1
        pltpu.make_async_copy(xt_hbm.at[pk1.at[0]], buf1, sem1).wait()
        _apply_ew(buf1, pk1)
        pltpu.sync_copy(buf1, acc_sh.at[pk1.at[1]], add=True)

        @pl.when(k < NB // 2 - 1)
        def _():
            pltpu.async_copy(pk_hbm.at[wid, j1 + 2], pk1, semp1)
            pltpu.make_async_copy(pk_hbm.at[wid, j0 + 2], pk0, semp0).wait()
            pltpu.async_copy(xt_hbm.at[pk0.at[0]], buf0, sem0)
            pltpu.make_async_copy(pk_hbm.at[wid, j1 + 2], pk1, semp1).wait()
            pltpu.async_copy(xt_hbm.at[pk1.at[0]], buf1, sem1)

        return 0

    lax.fori_loop(0, NB // 2, _outer, 0)
    plsc.subcore_barrier()

    pltpu.sync_copy(acc_sh.at[pl.ds(sid * 640, 640)],
                    out_hbm.at[cid, pl.ds(sid * 640, 640)])


def _sc_edges(xt_p, pk, we_f, be_f):
    mesh = plsc.VectorSubcoreMesh(core_axis_name="c", subcore_axis_name="s")
    f = pl.kernel(
        _sc_body,
        out_type=jax.ShapeDtypeStruct((2, ACCR, H), jnp.float32),
        mesh=mesh,
        compiler_params=pltpu.CompilerParams(needs_layout_passes=False),
        scratch_types=[
            pltpu.VMEM((3, EB), jnp.int32),
            pltpu.VMEM((3, EB), jnp.int32),
            pltpu.VMEM((H,), jnp.float32),
            pltpu.VMEM((H,), jnp.float32),
            pltpu.VMEM((EB, H), jnp.float32),
            pltpu.VMEM((EB, H), jnp.float32),
            pltpu.VMEM_SHARED((ACCR, H), jnp.float32),
            pltpu.SemaphoreType.DMA,
            pltpu.SemaphoreType.DMA,
            pltpu.SemaphoreType.DMA,
            pltpu.SemaphoreType.DMA,
        ],
    )
    return f(xt_p, pk, we_f, be_f)


def _post_body(a0_ref, a1_ref, xt_ref, we_ref, be_ref, cb_ref, g2_ref,
               b2_ref, wa1_ref, ba1_ref, wa2_ref, ba2_ref, wg1_ref, bg1_ref,
               wg2_ref, bg2_ref, wh1_ref, bh1_ref, gh_ref, bh_ref, wh2_ref,
               bh2_ref, wh3_ref, bh3_ref, out_ref):
    ewl = _sigmoid(we_ref[...] + be_ref[...])
    acc = (a0_ref[0] + a1_ref[0] + xt_ref[...] * ewl + cb_ref[...])
    o = _ln(_elu(acc), g2_ref[...], b2_ref[...])

    t1 = jnp.tanh(jnp.dot(o, wa1_ref[...],
                          preferred_element_type=jnp.float32) + ba1_ref[...])
    scol = jnp.dot(t1, wa2_ref[...],
                   preferred_element_type=jnp.float32) + ba2_ref[...]

    ri = lax.broadcasted_iota(jnp.int32, (RP, RP), 0)
    rj = lax.broadcasted_iota(jnp.int32, (RP, RP), 1)
    valid = (lax.broadcasted_iota(jnp.int32, (RP, 1), 0) < PER)
    s_eff = jnp.where(valid, scol, _NEG)
    eye = (ri == rj).astype(jnp.float32)
    srow = lax.dot_general(s_eff, eye, (((0,), (0,)), ((), ())),
                           preferred_element_type=jnp.float32)
    gt = (srow > s_eff).astype(jnp.float32)
    tie = ((srow == s_eff) & (rj < ri)).astype(jnp.float32)
    rank = jnp.sum(gt + tie, axis=1, keepdims=True)
    sel = rank < KP

    xp = o * _sigmoid(scol)
    tg = jnp.dot(jnp.tanh(jnp.dot(xp, wg1_ref[...],
                                  preferred_element_type=jnp.float32)
                          + bg1_ref[...]), wg2_ref[...],
                 preferred_element_type=jnp.float32) + bg2_ref[...]
    mt = jnp.where(sel, tg, _NEG)
    tmax = jnp.max(mt, axis=0, keepdims=True)
    a = jnp.where(sel, jnp.exp(tg - tmax), 0.0)
    denom = jnp.sum(a, axis=0, keepdims=True)
    xg = lax.dot_general(a, xp, (((0,), (0,)), ((), ())),
                         preferred_element_type=jnp.float32) / denom

    h1 = _elu(_ln(jnp.dot(xg, wh1_ref[...],
                          preferred_element_type=jnp.float32) + bh1_ref[...],
                  gh_ref[...], bh_ref[...]))
    h2 = _elu(jnp.dot(h1, wh2_ref[...],
                      preferred_element_type=jnp.float32) + bh2_ref[...])
    pr = jnp.dot(h2, wh3_ref[...],
                 preferred_element_type=jnp.float32) + bh3_ref[...]
    out_ref[...] = jnp.broadcast_to(pr.reshape(1, 1, 1), (1, 8, H))


def _post(accs, xt_p, we, be, cb, g2, b2, wa1, ba1, wa2, ba2, wg1, bg1,
          wg2, bg2, wh1, bh1, gh, bh, wh2, bh2, wh3, bh3):
    full = lambda *shape: pl.BlockSpec(shape, lambda g: (0,) * len(shape))
    return pl.pallas_call(
        _post_body,
        grid=(NG,),
        in_specs=[
            pl.BlockSpec((1, RP, H), lambda g: (0, g, 0)),
            pl.BlockSpec((1, RP, H), lambda g: (1, g, 0)),
            pl.BlockSpec((RP, H), lambda g: (g, 0)),
            full(1, H), full(1, H), full(1, H), full(1, H), full(1, H),
            full(H, H), full(1, H), full(H, 1), full(1, 1),
            full(H, H), full(1, H), full(H, 1), full(1, 1),
            full(H, H), full(1, H), full(1, H), full(1, H),
            full(H, H // 2), full(1, H // 2), full(H // 2, 1), full(1, 1),
        ],
        out_specs=pl.BlockSpec((1, 8, H), lambda g: (g, 0, 0)),
        out_shape=jax.ShapeDtypeStruct((NG, 8, H), jnp.float32),
    )(accs, accs, xt_p, we, be, cb, g2, b2, wa1, ba1, wa2, ba2, wg1, bg1,
      wg2, bg2, wh1, bh1, gh, bh, wh2, bh2, wh3, bh3)


def kernel(x, edge_index, edge_attr, batch, W_in, b_in, ln1_g, ln1_b,
           basis_kernels, roi_community, W_e, b_e, conv_bias, ln2_g, ln2_b,
           Wa1, ba1, Wa2, ba2, Wg1, bg1, Wg2, bg2, Wh1, bh1, lnh_g, lnh_b,
           Wh2, bh2, Wh3, bh3):
    f32 = jnp.float32
    row = lambda v: v.reshape(1, -1).astype(f32)

    x_p = jnp.pad(x.reshape(NG, PER, IN),
                  ((0, 0), (0, RP - PER), (0, INP - IN))).reshape(NP, INP)
    win_p = jnp.pad(W_in, ((0, INP - IN), (0, 0)))
    roi_p = jnp.pad(roi_community, ((0, RP - PER), (0, 0)))

    xt_p = _pre(x_p, win_p, row(b_in), row(ln1_g), row(ln1_b), roi_p,
                basis_kernels)

    src = edge_index[0]
    dst = edge_index[1]
    src_r = src + (RP - PER) * (src // PER)
    dst_r = dst + (RP - PER) * (dst // PER)
    pad = EP - E
    src_r = jnp.concatenate([src_r, jnp.zeros((pad,), jnp.int32)])
    dst_r = jnp.concatenate([dst_r, jnp.full((pad,), DUMMY, jnp.int32)])
    ea_r = jnp.concatenate([edge_attr.reshape(-1),
                            jnp.zeros((pad,), f32)])
    ea_bits = lax.bitcast_convert_type(ea_r, jnp.int32)
    pk = jnp.stack([src_r.reshape(NW, NB, EB), dst_r.reshape(NW, NB, EB),
                    ea_bits.reshape(NW, NB, EB)], axis=2)

    accs = _sc_edges(xt_p, pk, W_e.reshape(-1).astype(f32),
                     b_e.astype(f32))

    predm = _post(accs,
                  xt_p, row(W_e), row(b_e), row(conv_bias), row(ln2_g),
                  row(ln2_b), Wa1, row(ba1), Wa2, ba2.reshape(1, 1), Wg1,
                  row(bg1), Wg2, bg2.reshape(1, 1), Wh1, row(bh1),
                  row(lnh_g), row(lnh_b), Wh2, row(bh2), Wh3,
                  bh3.reshape(1, 1))
    return predm[:, 0, 0]


# speedup vs baseline: 4.0107x; 1.0004x over previous
import functools

import jax
import jax.numpy as jnp
from jax import lax
from jax.experimental import pallas as pl
from jax.experimental.pallas import tpu as pltpu
from jax.experimental.pallas import tpu_sc as plsc

N = 9916
NG = 37
PER = 268
E = 317312
IN = 268
H = 128
C = 7
KP = 214

RP = 272
INP = 272
NP = NG * RP
ACCR = 10240
DUMMY = 10080

NW = 32
EB = 128
NB = 78
EP = NW * NB * EB

_NEG = -3.0e38


def _ln(x, g, b, eps=1e-5):
    m = x.mean(-1, keepdims=True)
    v = ((x - m) ** 2).mean(-1, keepdims=True)
    return (x - m) / jnp.sqrt(v + eps) * g + b


def _elu(x):
    return jnp.where(x > 0, x, jnp.exp(x) - 1.0)


def _sigmoid(x):
    return 1.0 / (1.0 + jnp.exp(-x))


def _pre_body(x_ref, win_ref, bin_ref, g1_ref, b1_ref, roi_ref, basis_ref,
              out_ref):
    xb = jnp.dot(x_ref[...], win_ref[...],
                 preferred_element_type=jnp.float32) + bin_ref[...]
    hb = _elu(_ln(xb, g1_ref[...], b1_ref[...]))
    roi = roi_ref[...]
    roi = roi - jnp.max(roi, -1, keepdims=True)
    er = jnp.exp(roi)
    cw = er / jnp.sum(er, -1, keepdims=True)
    acc = jnp.zeros((RP, H), jnp.float32)
    for c in range(C):
        acc = acc + jnp.dot(hb * cw[:, c:c + 1], basis_ref[c],
                            preferred_element_type=jnp.float32)
    out_ref[...] = acc


def _pre(x_p, win_p, b_in, ln1_g, ln1_b, roi_p, basis):
    return pl.pallas_call(
        _pre_body,
        grid=(NG,),
        in_specs=[
            pl.BlockSpec((RP, INP), lambda g: (g, 0)),
            pl.BlockSpec((INP, H), lambda g: (0, 0)),
            pl.BlockSpec((1, H), lambda g: (0, 0)),
            pl.BlockSpec((1, H), lambda g: (0, 0)),
            pl.BlockSpec((1, H), lambda g: (0, 0)),
            pl.BlockSpec((RP, C), lambda g: (0, 0)),
            pl.BlockSpec((C, H, H), lambda g: (0, 0, 0)),
        ],
        out_specs=pl.BlockSpec((RP, H), lambda g: (g, 0)),
        out_shape=jax.ShapeDtypeStruct((NP, H), jnp.float32),
    )(x_p, win_p, b_in, ln1_g, ln1_b, roi_p, basis)


def _sc_body(xt_hbm, pk_hbm, we_hbm, be_hbm,
             out_hbm,
             pk0, pk1, we_v, be_v, buf0, buf1, acc_sh,
             sem0, sem1, semp0, semp1):
    cid = lax.axis_index("c")
    sid = lax.axis_index("s")
    wid = cid * 16 + sid

    pltpu.sync_copy(we_hbm, we_v)
    pltpu.sync_copy(be_hbm, be_v)

    nw = [-(we_v[pl.ds(f * 16, 16)]) for f in range(H // 16)]
    nb = [-(be_v[pl.ds(f * 16, 16)]) for f in range(H // 16)]

    zero16 = jnp.zeros((16,), jnp.float32)

    def _zrow(i, _):
        for f in range(H // 16):
            buf0[i, pl.ds(f * 16, 16)] = zero16
        return 0

    lax.fori_loop(0, EB, _zrow, 0)
    for k in range(640 // EB):
        pltpu.sync_copy(buf0, acc_sh.at[pl.ds(sid * 640 + k * EB, EB)])
    plsc.subcore_barrier()

    def _apply_ew(buf, pk):
        def _edge(e, _):
            e16 = jnp.full((16,), e, jnp.int32)
            two16 = jnp.full((16,), 2, jnp.int32)
            eab = plsc.bitcast(plsc.load_gather(pk, [two16, e16]),
                               jnp.float32)
            for f in range(H // 16):
                s = 1.0 / (1.0 + jnp.exp(eab * nw[f] + nb[f]))
                buf[e, pl.ds(f * 16, 16)] = buf[e, pl.ds(f * 16, 16)] * s
            return 0

        lax.fori_loop(0, EB, _edge, 0)

    pltpu.sync_copy(pk_hbm.at[wid, 0], pk0)
    pltpu.sync_copy(pk_hbm.at[wid, 1], pk1)
    pltpu.async_copy(xt_hbm.at[pk0.at[0]], buf0, sem0)
    pltpu.async_copy(xt_hbm.at[pk1.at[0]], buf1, sem1)

    def _outer(k, _):
        j0 = 2 * k
        pltpu.make_async_copy(xt_hbm.at[pk0.at[0]], buf0, sem0).wait()
        _apply_ew(buf0, pk0)
        pltpu.sync_copy(buf0, acc_sh.at[pk0.at[1]], add=True)

        @pl.when(k < NB // 2 - 1)
        def _():
            pltpu.async_copy(pk_hbm.at[wid, j0 + 2], pk0, semp0)

        j1 = 2 * k + 1
        pltpu.make_async_copy(xt_hbm.at[pk1.at[0]], buf1, sem1).wait()
        _apply_ew(buf1, pk1)
        pltpu.sync_copy(buf1, acc_sh.at[pk1.at[1]], add=True)

        @pl.when(k < NB // 2 - 1)
        def _():
            pltpu.async_copy(pk_hbm.at[wid, j1 + 2], pk1, semp1)
            pltpu.make_async_copy(pk_hbm.at[wid, j0 + 2], pk0, semp0).wait()
            pltpu.async_copy(xt_hbm.at[pk0.at[0]], buf0, sem0)
            pltpu.make_async_copy(pk_hbm.at[wid, j1 + 2], pk1, semp1).wait()
            pltpu.async_copy(xt_hbm.at[pk1.at[0]], buf1, sem1)

        return 0

    lax.fori_loop(0, NB // 2, _outer, 0)
    plsc.subcore_barrier()

    pltpu.sync_copy(acc_sh.at[pl.ds(sid * 640, 640)],
                    out_hbm.at[cid, pl.ds(sid * 640, 640)])


def _sc_edges(xt_p, pk, we_f, be_f):
    mesh = plsc.VectorSubcoreMesh(core_axis_name="c", subcore_axis_name="s")
    f = pl.kernel(
        _sc_body,
        out_type=jax.ShapeDtypeStruct((2, ACCR, H), jnp.float32),
        mesh=mesh,
        compiler_params=pltpu.CompilerParams(needs_layout_passes=False,
                                             use_tc_tiling_on_sc=True),
        scratch_types=[
            pltpu.VMEM((3, EB), jnp.int32),
            pltpu.VMEM((3, EB), jnp.int32),
            pltpu.VMEM((H,), jnp.float32),
            pltpu.VMEM((H,), jnp.float32),
            pltpu.VMEM((EB, H), jnp.float32),
            pltpu.VMEM((EB, H), jnp.float32),
            pltpu.VMEM_SHARED((ACCR, H), jnp.float32),
            pltpu.SemaphoreType.DMA,
            pltpu.SemaphoreType.DMA,
            pltpu.SemaphoreType.DMA,
            pltpu.SemaphoreType.DMA,
        ],
    )
    return f(xt_p, pk, we_f, be_f)


def _post_body(a0_ref, a1_ref, xt_ref, we_ref, be_ref, cb_ref, g2_ref,
               b2_ref, wa1_ref, ba1_ref, wa2_ref, ba2_ref, wg1_ref, bg1_ref,
               wg2_ref, bg2_ref, wh1_ref, bh1_ref, gh_ref, bh_ref, wh2_ref,
               bh2_ref, wh3_ref, bh3_ref, out_ref):
    ewl = _sigmoid(we_ref[...] + be_ref[...])
    acc = (a0_ref[0] + a1_ref[0] + xt_ref[...] * ewl + cb_ref[...])
    o = _ln(_elu(acc), g2_ref[...], b2_ref[...])

    t1 = jnp.tanh(jnp.dot(o, wa1_ref[...],
                          preferred_element_type=jnp.float32) + ba1_ref[...])
    scol = jnp.dot(t1, wa2_ref[...],
                   preferred_element_type=jnp.float32) + ba2_ref[...]

    ri = lax.broadcasted_iota(jnp.int32, (RP, RP), 0)
    rj = lax.broadcasted_iota(jnp.int32, (RP, RP), 1)
    valid = (lax.broadcasted_iota(jnp.int32, (RP, 1), 0) < PER)
    s_eff = jnp.where(valid, scol, _NEG)
    eye = (ri == rj).astype(jnp.float32)
    srow = lax.dot_general(s_eff, eye, (((0,), (0,)), ((), ())),
                           preferred_element_type=jnp.float32)
    gt = (srow > s_eff).astype(jnp.float32)
    tie = ((srow == s_eff) & (rj < ri)).astype(jnp.float32)
    rank = jnp.sum(gt + tie, axis=1, keepdims=True)
    sel = rank < KP

    xp = o * _sigmoid(scol)
    tg = jnp.dot(jnp.tanh(jnp.dot(xp, wg1_ref[...],
                                  preferred_element_type=jnp.float32)
                          + bg1_ref[...]), wg2_ref[...],
                 preferred_element_type=jnp.float32) + bg2_ref[...]
    mt = jnp.where(sel, tg, _NEG)
    tmax = jnp.max(mt, axis=0, keepdims=True)
    a = jnp.where(sel, jnp.exp(tg - tmax), 0.0)
    denom = jnp.sum(a, axis=0, keepdims=True)
    xg = lax.dot_general(a, xp, (((0,), (0,)), ((), ())),
                         preferred_element_type=jnp.float32) / denom

    h1 = _elu(_ln(jnp.dot(xg, wh1_ref[...],
                          preferred_element_type=jnp.float32) + bh1_ref[...],
                  gh_ref[...], bh_ref[...]))
    h2 = _elu(jnp.dot(h1, wh2_ref[...],
                      preferred_element_type=jnp.float32) + bh2_ref[...])
    pr = jnp.dot(h2, wh3_ref[...],
                 preferred_element_type=jnp.float32) + bh3_ref[...]
    out_ref[...] = jnp.broadcast_to(pr.reshape(1, 1, 1), (1, 8, H))


def _post(accs, xt_p, we, be, cb, g2, b2, wa1, ba1, wa2, ba2, wg1, bg1,
          wg2, bg2, wh1, bh1, gh, bh, wh2, bh2, wh3, bh3):
    full = lambda *shape: pl.BlockSpec(shape, lambda g: (0,) * len(shape))
    return pl.pallas_call(
        _post_body,
        grid=(NG,),
        in_specs=[
            pl.BlockSpec((1, RP, H), lambda g: (0, g, 0)),
            pl.BlockSpec((1, RP, H), lambda g: (1, g, 0)),
            pl.BlockSpec((RP, H), lambda g: (g, 0)),
            full(1, H), full(1, H), full(1, H), full(1, H), full(1, H),
            full(H, H), full(1, H), full(H, 1), full(1, 1),
            full(H, H), full(1, H), full(H, 1), full(1, 1),
            full(H, H), full(1, H), full(1, H), full(1, H),
            full(H, H // 2), full(1, H // 2), full(H // 2, 1), full(1, 1),
        ],
        out_specs=pl.BlockSpec((1, 8, H), lambda g: (g, 0, 0)),
        out_shape=jax.ShapeDtypeStruct((NG, 8, H), jnp.float32),
    )(accs, accs, xt_p, we, be, cb, g2, b2, wa1, ba1, wa2, ba2, wg1, bg1,
      wg2, bg2, wh1, bh1, gh, bh, wh2, bh2, wh3, bh3)


def kernel(x, edge_index, edge_attr, batch, W_in, b_in, ln1_g, ln1_b,
           basis_kernels, roi_community, W_e, b_e, conv_bias, ln2_g, ln2_b,
           Wa1, ba1, Wa2, ba2, Wg1, bg1, Wg2, bg2, Wh1, bh1, lnh_g, lnh_b,
           Wh2, bh2, Wh3, bh3):
    f32 = jnp.float32
    row = lambda v: v.reshape(1, -1).astype(f32)

    x_p = jnp.pad(x.reshape(NG, PER, IN),
                  ((0, 0), (0, RP - PER), (0, INP - IN))).reshape(NP, INP)
    win_p = jnp.pad(W_in, ((0, INP - IN), (0, 0)))
    roi_p = jnp.pad(roi_community, ((0, RP - PER), (0, 0)))

    xt_p = _pre(x_p, win_p, row(b_in), row(ln1_g), row(ln1_b), roi_p,
                basis_kernels)

    src = edge_index[0]
    dst = edge_index[1]
    src_r = src + (RP - PER) * (src // PER)
    dst_r = dst + (RP - PER) * (dst // PER)
    pad = EP - E
    src_r = jnp.concatenate([src_r, jnp.zeros((pad,), jnp.int32)])
    dst_r = jnp.concatenate([dst_r, jnp.full((pad,), DUMMY, jnp.int32)])
    ea_r = jnp.concatenate([edge_attr.reshape(-1),
                            jnp.zeros((pad,), f32)])
    ea_bits = lax.bitcast_convert_type(ea_r, jnp.int32)
    pk = jnp.stack([src_r.reshape(NW, NB, EB), dst_r.reshape(NW, NB, EB),
                    ea_bits.reshape(NW, NB, EB)], axis=2)

    accs = _sc_edges(xt_p, pk, W_e.reshape(-1).astype(f32),
                     b_e.astype(f32))

    predm = _post(accs,
                  xt_p, row(W_e), row(b_e), row(conv_bias), row(ln2_g),
                  row(ln2_b), Wa1, row(ba1), Wa2, ba2.reshape(1, 1), Wg1,
                  row(bg1), Wg2, bg2.reshape(1, 1), Wh1, row(bh1),
                  row(lnh_g), row(lnh_b), Wh2, row(bh2), Wh3,
                  bh3.reshape(1, 1))
    return predm[:, 0, 0]


# speedup vs baseline: 5.1642x; 1.2876x over previous
import functools

import jax
import jax.numpy as jnp
from jax import lax
from jax.experimental import pallas as pl
from jax.experimental.pallas import tpu as pltpu
from jax.experimental.pallas import tpu_sc as plsc

N = 9916
NG = 37
PER = 268
E = 317312
IN = 268
H = 128
C = 7
KP = 214

RP = 272
INP = 272
NP = NG * RP
ACCR = 10240
DUMMY = 10080

NW = 32
EB = 80
NB = 124
EP = NW * NB * EB
ET = NB * EB

_NEG = -3.0e38


def _ln(x, g, b, eps=1e-5):
    m = x.mean(-1, keepdims=True)
    v = ((x - m) ** 2).mean(-1, keepdims=True)
    return (x - m) / jnp.sqrt(v + eps) * g + b


def _elu(x):
    return jnp.where(x > 0, x, jnp.exp(x) - 1.0)


def _sigmoid(x):
    return 1.0 / (1.0 + jnp.exp(-x))


def _pre_body(x_ref, win_ref, bin_ref, g1_ref, b1_ref, roi_ref, basis_ref,
              out_ref):
    xb = jnp.dot(x_ref[0], win_ref[...],
                 preferred_element_type=jnp.float32) + bin_ref[...]
    hb = _elu(_ln(xb, g1_ref[...], b1_ref[...]))
    roi = roi_ref[...]
    roi = roi - jnp.max(roi, -1, keepdims=True)
    er = jnp.exp(roi)
    cw = er / jnp.sum(er, -1, keepdims=True)
    acc = jnp.zeros((PER, H), jnp.float32)
    for c in range(C):
        acc = acc + jnp.dot(hb * cw[:, c:c + 1], basis_ref[c],
                            preferred_element_type=jnp.float32)
    out_ref[...] = jnp.concatenate(
        [acc, jnp.zeros((RP - PER, H), jnp.float32)], axis=0)


def _pre(x3, w_in, b_in, ln1_g, ln1_b, roi, basis):
    return pl.pallas_call(
        _pre_body,
        grid=(NG,),
        in_specs=[
            pl.BlockSpec((1, PER, IN), lambda g: (g, 0, 0)),
            pl.BlockSpec((IN, H), lambda g: (0, 0)),
            pl.BlockSpec((1, H), lambda g: (0, 0)),
            pl.BlockSpec((1, H), lambda g: (0, 0)),
            pl.BlockSpec((1, H), lambda g: (0, 0)),
            pl.BlockSpec((PER, C), lambda g: (0, 0)),
            pl.BlockSpec((C, H, H), lambda g: (0, 0, 0)),
        ],
        out_specs=pl.BlockSpec((RP, H), lambda g: (g, 0)),
        out_shape=jax.ShapeDtypeStruct((NP, H), jnp.float32),
    )(x3, w_in, b_in, ln1_g, ln1_b, roi, basis)


def _sc_body(xt_hbm, src_hbm, dst_hbm, ea_hbm, we_hbm, be_hbm,
             out_hbm,
             s0, s1, d0, d1, a0, a1, we_v, be_v, buf0, buf1, acc_sh,
             sem0, sem1, semi0, semi1):
    cid = lax.axis_index("c")
    sid = lax.axis_index("s")
    wid = cid * 16 + sid
    base = wid * ET

    pltpu.sync_copy(we_hbm, we_v)
    pltpu.sync_copy(be_hbm, be_v)

    nw = [-(we_v[pl.ds(f * 16, 16)]) for f in range(H // 16)]
    nb = [-(be_v[pl.ds(f * 16, 16)]) for f in range(H // 16)]

    zero16 = jnp.zeros((16,), jnp.float32)

    def _zrow(i, _):
        for f in range(H // 16):
            buf0[i, pl.ds(f * 16, 16)] = zero16
        return 0

    lax.fori_loop(0, EB, _zrow, 0)
    for k in range(640 // EB):
        pltpu.sync_copy(buf0, acc_sh.at[pl.ds(sid * 640 + k * EB, EB)])
    plsc.subcore_barrier()

    def _fetch_idx(j, sv, dv, av, sem):
        off = base + j * EB
        c0 = pltpu.async_copy(src_hbm.at[pl.ds(off, EB)], sv, sem)
        pltpu.async_copy(dst_hbm.at[pl.ds(off, EB)], dv, sem)
        pltpu.async_copy(ea_hbm.at[pl.ds(off, EB)], av, sem)
        return c0

    def _wait_idx(j, sv, dv, av, sem):
        off = base + j * EB
        pltpu.make_async_copy(src_hbm.at[pl.ds(off, EB)], sv, sem).wait()
        pltpu.make_async_copy(dst_hbm.at[pl.ds(off, EB)], dv, sem).wait()
        pltpu.make_async_copy(ea_hbm.at[pl.ds(off, EB)], av, sem).wait()

    def _apply_ew(buf, av):
        def _edge(e, _):
            eab = plsc.load_gather(av, [jnp.full((16,), e, jnp.int32)])
            for f in range(H // 16):
                s = 1.0 / (1.0 + jnp.exp(eab * nw[f] + nb[f]))
                buf[e, pl.ds(f * 16, 16)] = buf[e, pl.ds(f * 16, 16)] * s
            return 0

        lax.fori_loop(0, EB, _edge, 0)

    _fetch_idx(0, s0, d0, a0, semi0)
    _wait_idx(0, s0, d0, a0, semi0)
    _fetch_idx(1, s1, d1, a1, semi1)
    _wait_idx(1, s1, d1, a1, semi1)
    pltpu.async_copy(xt_hbm.at[s0], buf0, sem0)
    pltpu.async_copy(xt_hbm.at[s1], buf1, sem1)

    def _outer(k, _):
        j0 = 2 * k
        pltpu.make_async_copy(xt_hbm.at[s0], buf0, sem0).wait()
        _apply_ew(buf0, a0)
        pltpu.sync_copy(buf0, acc_sh.at[d0], add=True)

        @pl.when(k < NB // 2 - 1)
        def _():
            _fetch_idx(j0 + 2, s0, d0, a0, semi0)

        j1 = 2 * k + 1
        pltpu.make_async_copy(xt_hbm.at[s1], buf1, sem1).wait()
        _apply_ew(buf1, a1)
        pltpu.sync_copy(buf1, acc_sh.at[d1], add=True)

        @pl.when(k < NB // 2 - 1)
        def _():
            _fetch_idx(j1 + 2, s1, d1, a1, semi1)
            _wait_idx(j0 + 2, s0, d0, a0, semi0)
            pltpu.async_copy(xt_hbm.at[s0], buf0, sem0)
            _wait_idx(j1 + 2, s1, d1, a1, semi1)
            pltpu.async_copy(xt_hbm.at[s1], buf1, sem1)

        return 0

    lax.fori_loop(0, NB // 2, _outer, 0)
    plsc.subcore_barrier()

    pltpu.sync_copy(acc_sh.at[pl.ds(sid * 640, 640)],
                    out_hbm.at[cid, pl.ds(sid * 640, 640)])


def _sc_edges(xt_p, src_r, dst_r, ea_r, we_f, be_f):
    mesh = plsc.VectorSubcoreMesh(core_axis_name="c", subcore_axis_name="s")
    f = pl.kernel(
        _sc_body,
        out_type=jax.ShapeDtypeStruct((2, ACCR, H), jnp.float32),
        mesh=mesh,
        compiler_params=pltpu.CompilerParams(needs_layout_passes=False,
                                             use_tc_tiling_on_sc=True),
        scratch_types=[
            pltpu.VMEM((EB,), jnp.int32),
            pltpu.VMEM((EB,), jnp.int32),
            pltpu.VMEM((EB,), jnp.int32),
            pltpu.VMEM((EB,), jnp.int32),
            pltpu.VMEM((EB,), jnp.float32),
            pltpu.VMEM((EB,), jnp.float32),
            pltpu.VMEM((H,), jnp.float32),
            pltpu.VMEM((H,), jnp.float32),
            pltpu.VMEM((EB, H), jnp.float32),
            pltpu.VMEM((EB, H), jnp.float32),
            pltpu.VMEM_SHARED((ACCR, H), jnp.float32),
            pltpu.SemaphoreType.DMA,
            pltpu.SemaphoreType.DMA,
            pltpu.SemaphoreType.DMA,
            pltpu.SemaphoreType.DMA,
        ],
    )
    return f(xt_p, src_r, dst_r, ea_r, we_f, be_f)


def _post_body(a0_ref, a1_ref, xt_ref, we_ref, be_ref, cb_ref, g2_ref,
               b2_ref, wa1_ref, ba1_ref, wa2_ref, ba2_ref, wg1_ref, bg1_ref,
               wg2_ref, bg2_ref, wh1_ref, bh1_ref, gh_ref, bh_ref, wh2_ref,
               bh2_ref, wh3_ref, bh3_ref, out_ref):
    ewl = _sigmoid(we_ref[...] + be_ref[...])
    acc = (a0_ref[0] + a1_ref[0] + xt_ref[...] * ewl + cb_ref[...])
    o = _ln(_elu(acc), g2_ref[...], b2_ref[...])

    t1 = jnp.tanh(jnp.dot(o, wa1_ref[...],
                          preferred_element_type=jnp.float32) + ba1_ref[...])
    scol = jnp.dot(t1, wa2_ref[...],
                   preferred_element_type=jnp.float32) + ba2_ref[...]

    ri = lax.broadcasted_iota(jnp.int32, (RP, RP), 0)
    rj = lax.broadcasted_iota(jnp.int32, (RP, RP), 1)
    valid = (lax.broadcasted_iota(jnp.int32, (RP, 1), 0) < PER)
    s_eff = jnp.where(valid, scol, _NEG)
    eye = (ri == rj).astype(jnp.float32)
    srow = lax.dot_general(s_eff, eye, (((0,), (0,)), ((), ())),
                           preferred_element_type=jnp.float32)
    gt = (srow > s_eff).astype(jnp.float32)
    tie = ((srow == s_eff) & (rj < ri)).astype(jnp.float32)
    rank = jnp.sum(gt + tie, axis=1, keepdims=True)
    sel = rank < KP

    xp = o * _sigmoid(scol)
    tg = jnp.dot(jnp.tanh(jnp.dot(xp, wg1_ref[...],
                                  preferred_element_type=jnp.float32)
                          + bg1_ref[...]), wg2_ref[...],
                 preferred_element_type=jnp.float32) + bg2_ref[...]
    mt = jnp.where(sel, tg, _NEG)
    tmax = jnp.max(mt, axis=0, keepdims=True)
    a = jnp.where(sel, jnp.exp(tg - tmax), 0.0)
    denom = jnp.sum(a, axis=0, keepdims=True)
    xg = lax.dot_general(a, xp, (((0,), (0,)), ((), ())),
                         preferred_element_type=jnp.float32) / denom

    h1 = _elu(_ln(jnp.dot(xg, wh1_ref[...],
                          preferred_element_type=jnp.float32) + bh1_ref[...],
                  gh_ref[...], bh_ref[...]))
    h2 = _elu(jnp.dot(h1, wh2_ref[...],
                      preferred_element_type=jnp.float32) + bh2_ref[...])
    pr = jnp.dot(h2, wh3_ref[...],
                 preferred_element_type=jnp.float32) + bh3_ref[...]
    out_ref[...] = jnp.broadcast_to(pr.reshape(1, 1, 1), (1, 8, H))


def _post(accs, xt_p, we, be, cb, g2, b2, wa1, ba1, wa2, ba2, wg1, bg1,
          wg2, bg2, wh1, bh1, gh, bh, wh2, bh2, wh3, bh3):
    full = lambda *shape: pl.BlockSpec(shape, lambda g: (0,) * len(shape))
    return pl.pallas_call(
        _post_body,
        grid=(NG,),
        in_specs=[
            pl.BlockSpec((1, RP, H), lambda g: (0, g, 0)),
            pl.BlockSpec((1, RP, H), lambda g: (1, g, 0)),
            pl.BlockSpec((RP, H), lambda g: (g, 0)),
            full(1, H), full(1, H), full(1, H), full(1, H), full(1, H),
            full(H, H), full(1, H), full(H, 1), full(1, 1),
            full(H, H), full(1, H), full(H, 1), full(1, 1),
            full(H, H), full(1, H), full(1, H), full(1, H),
            full(H, H // 2), full(1, H // 2), full(H // 2, 1), full(1, 1),
        ],
        out_specs=pl.BlockSpec((1, 8, H), lambda g: (g, 0, 0)),
        out_shape=jax.ShapeDtypeStruct((NG, 8, H), jnp.float32),
    )(accs, accs, xt_p, we, be, cb, g2, b2, wa1, ba1, wa2, ba2, wg1, bg1,
      wg2, bg2, wh1, bh1, gh, bh, wh2, bh2, wh3, bh3)


def kernel(x, edge_index, edge_attr, batch, W_in, b_in, ln1_g, ln1_b,
           basis_kernels, roi_community, W_e, b_e, conv_bias, ln2_g, ln2_b,
           Wa1, ba1, Wa2, ba2, Wg1, bg1, Wg2, bg2, Wh1, bh1, lnh_g, lnh_b,
           Wh2, bh2, Wh3, bh3):
    f32 = jnp.float32
    row = lambda v: v.reshape(1, -1).astype(f32)

    xt_p = _pre(x.reshape(NG, PER, IN), W_in, row(b_in), row(ln1_g),
                row(ln1_b), roi_community, basis_kernels)

    pad = EP - E
    src = jnp.concatenate([edge_index[0], jnp.zeros((pad,), jnp.int32)])
    dst = jnp.concatenate([edge_index[1], jnp.full((pad,), DUMMY, jnp.int32)])
    src_r = src + (RP - PER) * (src // PER)
    dst_r = dst + (RP - PER) * (dst // PER)
    ea_r = jnp.concatenate([edge_attr.reshape(-1), jnp.zeros((pad,), f32)])

    accs = _sc_edges(xt_p, src_r, dst_r, ea_r, W_e.reshape(-1).astype(f32),
                     b_e.astype(f32))

    predm = _post(accs,
                  xt_p, row(W_e), row(b_e), row(conv_bias), row(ln2_g),
                  row(ln2_b), Wa1, row(ba1), Wa2, ba2.reshape(1, 1), Wg1,
                  row(bg1), Wg2, bg2.reshape(1, 1), Wh1, row(bh1),
                  row(lnh_g), row(lnh_b), Wh2, row(bh2), Wh3,
                  bh3.reshape(1, 1))
    return predm[:, 0, 0]
